# baseline (speedup 1.0000x reference)
import functools
import jax
import jax.numpy as jnp
from jax import lax
from jax.experimental import pallas as pl
from jax.experimental.pallas import tpu as pltpu
from jax.experimental.pallas import tpu_sc as plsc

_B, _H, _V, _D = 8, 16, 8192, 64
_N = 4096
_NC, _NS = 2, 16
_NW = _NC * _NS
_R = _B * _H * _D
_RPW = _R // _NW
_RPB = 4
_NB = _RPW // _RPB


def _sc_gather(tt, idx):
    mesh = plsc.VectorSubcoreMesh(core_axis_name="c", subcore_axis_name="s")

    @functools.partial(
        pl.kernel,
        out_type=jax.ShapeDtypeStruct((_R, _N), jnp.float32),
        mesh=mesh,
        compiler_params=pltpu.CompilerParams(needs_layout_passes=False),
        scratch_types=[
            pltpu.VMEM((_N,), jnp.int32),
            pltpu.VMEM((_RPB, _V), jnp.float32),
            pltpu.VMEM((_RPB, _V), jnp.float32),
            pltpu.VMEM((_RPB, _N), jnp.float32),
            pltpu.VMEM((_RPB, _N), jnp.float32),
            pltpu.SemaphoreType.DMA,
            pltpu.SemaphoreType.DMA,
            pltpu.SemaphoreType.DMA,
            pltpu.SemaphoreType.DMA,
        ],
    )
    def body(t_hbm, idx_hbm, out_hbm,
             idx_v, rb0, rb1, cb0, cb1, gs0, gs1, ss0, ss1):
        cid = lax.axis_index("c")
        sid = lax.axis_index("s")
        wid = sid * _NC + cid
        base = wid * _RPW
        pltpu.sync_copy(idx_hbm, idx_v)

        def issue_read(b, rb, sem):
            pltpu.async_copy(
                t_hbm.at[pl.ds(base + b * _RPB, _RPB)], rb, sem
            )

        def wait_read(rb, sem):
            pltpu.make_async_copy(
                t_hbm.at[pl.ds(0, _RPB)], rb, sem
            ).wait()

        kvs = [jnp.full((16,), k, jnp.int32) for k in range(_RPB)]

        def gather(rb, cb):
            @plsc.parallel_loop(0, _N // 16, 1, unroll=16)
            def grp(g):
                iv = idx_v[pl.ds(g * 16, 16)]
                for k in range(_RPB):
                    cb[k, pl.ds(g * 16, 16)] = plsc.load_gather(
                        rb, [kvs[k], iv]
                    )

        def issue_store(b, cb, sem):
            pltpu.async_copy(
                cb, out_hbm.at[pl.ds(base + b * _RPB, _RPB)], sem
            )

        def wait_store(cb, sem):
            pltpu.make_async_copy(
                cb, out_hbm.at[pl.ds(0, _RPB)], sem
            ).wait()

        issue_read(0, rb0, gs0)

        def loop(i, carry):
            b0 = 2 * i
            b1 = 2 * i + 1
            issue_read(b1, rb1, gs1)
            wait_read(rb0, gs0)

            @pl.when(i > 0)
            def _():
                wait_store(cb0, ss0)

            gather(rb0, cb0)
            issue_store(b0, cb0, ss0)

            @pl.when(i < _NB // 2 - 1)
            def _():
                issue_read(b0 + 2, rb0, gs0)

            wait_read(rb1, gs1)

            @pl.when(i > 0)
            def _():
                wait_store(cb1, ss1)

            gather(rb1, cb1)
            issue_store(b1, cb1, ss1)
            return carry

        lax.fori_loop(0, _NB // 2, loop, 0)
        wait_store(cb0, ss0)
        wait_store(cb1, ss1)

    return body(tt, idx)


def kernel(t, idx):
    tt = jnp.transpose(t, (0, 1, 3, 2)).reshape(_R, _V)
    out_t = _sc_gather(tt, idx.astype(jnp.int32))
    return jnp.transpose(out_t.reshape(_B, _H, _D, _N), (0, 1, 3, 2))


# speedup vs baseline: 1.0251x; 1.0251x over previous
import functools
import jax
import jax.numpy as jnp
from jax import lax
from jax.experimental import pallas as pl
from jax.experimental.pallas import tpu as pltpu
from jax.experimental.pallas import tpu_sc as plsc

_B, _H, _V, _D = 8, 16, 8192, 64
_N = 4096
_NC, _NS = 2, 16
_NW = _NC * _NS
_R = _B * _H * _D
_RPW = _R // _NW
_RPB = 2
_NB = _RPW // _RPB
_RING = 4


def _sc_gather(tt, idx):
    mesh = plsc.VectorSubcoreMesh(core_axis_name="c", subcore_axis_name="s")

    @functools.partial(
        pl.kernel,
        out_type=jax.ShapeDtypeStruct((_R, _N), jnp.float32),
        mesh=mesh,
        compiler_params=pltpu.CompilerParams(needs_layout_passes=False),
        scratch_types=(
            [pltpu.VMEM((_N,), jnp.int32)]
            + [pltpu.VMEM((_RPB, _V), jnp.float32) for _ in range(_RING)]
            + [pltpu.VMEM((_RPB, _N), jnp.float32) for _ in range(2)]
            + [pltpu.SemaphoreType.DMA for _ in range(_RING + 2)]
        ),
    )
    def body(t_hbm, idx_hbm, out_hbm, idx_v, *bufs):
        rbs = bufs[:_RING]
        cbs = bufs[_RING:_RING + 2]
        gss = bufs[_RING + 2:2 * _RING + 2]
        sss = bufs[2 * _RING + 2:]
        cid = lax.axis_index("c")
        sid = lax.axis_index("s")
        wid = sid * _NC + cid
        base = wid * _RPW
        pltpu.sync_copy(idx_hbm, idx_v)

        def issue_read(b, rb, sem):
            pltpu.async_copy(
                t_hbm.at[pl.ds(base + b * _RPB, _RPB)], rb, sem
            )

        def wait_read(rb, sem):
            pltpu.make_async_copy(
                t_hbm.at[pl.ds(0, _RPB)], rb, sem
            ).wait()

        kvs = [jnp.full((16,), k, jnp.int32) for k in range(_RPB)]

        def gather(rb, cb):
            @plsc.parallel_loop(0, _N // 16, 1, unroll=8)
            def grp(g):
                iv = idx_v[pl.ds(g * 16, 16)]
                for k in range(_RPB):
                    cb[k, pl.ds(g * 16, 16)] = plsc.load_gather(
                        rb, [kvs[k], iv]
                    )

        def issue_store(b, cb, sem):
            pltpu.async_copy(
                cb, out_hbm.at[pl.ds(base + b * _RPB, _RPB)], sem
            )

        def wait_store(cb, sem):
            pltpu.make_async_copy(
                cb, out_hbm.at[pl.ds(0, _RPB)], sem
            ).wait()

        for j in range(_RING):
            issue_read(j, rbs[j], gss[j])

        def loop(i, carry):
            for j in range(_RING):
                b = _RING * i + j
                wait_read(rbs[j], gss[j])

                @pl.when((i > 0) | (j >= 2))
                def _():
                    wait_store(cbs[j % 2], sss[j % 2])

                gather(rbs[j], cbs[j % 2])
                issue_store(b, cbs[j % 2], sss[j % 2])

                @pl.when(i < _NB // _RING - 1)
                def _():
                    issue_read(b + _RING, rbs[j], gss[j])

            return carry

        lax.fori_loop(0, _NB // _RING, loop, 0)
        wait_store(cbs[0], sss[0])
        wait_store(cbs[1], sss[1])

    return body(tt, idx)


def kernel(t, idx):
    tt = jnp.transpose(t, (0, 1, 3, 2)).reshape(_R, _V)
    out_t = _sc_gather(tt, idx.astype(jnp.int32))
    return jnp.transpose(out_t.reshape(_B, _H, _D, _N), (0, 1, 3, 2))


# speedup vs baseline: 1.0261x; 1.0010x over previous
import functools
import jax
import jax.numpy as jnp
from jax import lax
from jax.experimental import pallas as pl
from jax.experimental.pallas import tpu as pltpu
from jax.experimental.pallas import tpu_sc as plsc

_B, _H, _V, _D = 8, 16, 8192, 64
_N = 4096
_NC, _NS = 2, 16
_NW = _NC * _NS
_R = _B * _H * _D
_RPW = _R // _NW
_RPB = 2
_NB = _RPW // _RPB
_RING = 4


def _sc_gather(tt, idx):
    mesh = plsc.VectorSubcoreMesh(core_axis_name="c", subcore_axis_name="s")

    @functools.partial(
        pl.kernel,
        out_type=jax.ShapeDtypeStruct((_R, _N), jnp.float32),
        mesh=mesh,
        compiler_params=pltpu.CompilerParams(needs_layout_passes=False),
        scratch_types=(
            [pltpu.VMEM((_N,), jnp.int32)]
            + [pltpu.VMEM((_RPB, _V), jnp.float32) for _ in range(_RING)]
            + [pltpu.VMEM((_RPB, _N), jnp.float32) for _ in range(2)]
            + [pltpu.SemaphoreType.DMA for _ in range(_RING + 2)]
        ),
    )
    def body(t_hbm, idx_hbm, out_hbm, idx_v, *bufs):
        rbs = bufs[:_RING]
        cbs = bufs[_RING:_RING + 2]
        gss = bufs[_RING + 2:2 * _RING + 2]
        sss = bufs[2 * _RING + 2:]
        cid = lax.axis_index("c")
        sid = lax.axis_index("s")
        wid = sid * _NC + cid
        base = wid * _RPW
        pltpu.sync_copy(idx_hbm, idx_v)

        def issue_read(b, rb, sem):
            pltpu.async_copy(
                t_hbm.at[pl.ds(base + b * _RPB, _RPB)], rb, sem
            )

        def wait_read(rb, sem):
            pltpu.make_async_copy(
                t_hbm.at[pl.ds(0, _RPB)], rb, sem
            ).wait()

        kvs = [jnp.full((16,), k, jnp.int32) for k in range(_RPB)]

        def gather(rb, cb):
            @plsc.parallel_loop(0, _N // 16, 1, unroll=4)
            def grp(g):
                iv = idx_v[pl.ds(g * 16, 16)]
                for k in range(_RPB):
                    cb[k, pl.ds(g * 16, 16)] = plsc.load_gather(
                        rb, [kvs[k], iv]
                    )

        def issue_store(b, cb, sem):
            pltpu.async_copy(
                cb, out_hbm.at[pl.ds(base + b * _RPB, _RPB)], sem
            )

        def wait_store(cb, sem):
            pltpu.make_async_copy(
                cb, out_hbm.at[pl.ds(0, _RPB)], sem
            ).wait()

        for j in range(_RING):
            issue_read(j, rbs[j], gss[j])

        def loop(i, carry):
            for j in range(_RING):
                b = _RING * i + j
                wait_read(rbs[j], gss[j])

                @pl.when((i > 0) | (j >= 2))
                def _():
                    wait_store(cbs[j % 2], sss[j % 2])

                gather(rbs[j], cbs[j % 2])
                issue_store(b, cbs[j % 2], sss[j % 2])

                @pl.when(i < _NB // _RING - 1)
                def _():
                    issue_read(b + _RING, rbs[j], gss[j])

            return carry

        lax.fori_loop(0, _NB // _RING, loop, 0)
        wait_store(cbs[0], sss[0])
        wait_store(cbs[1], sss[1])

    return body(tt, idx)


def kernel(t, idx):
    tt = jnp.transpose(t, (0, 1, 3, 2)).reshape(_R, _V)
    out_t = _sc_gather(tt, idx.astype(jnp.int32))
    return jnp.transpose(out_t.reshape(_B, _H, _D, _N), (0, 1, 3, 2))
